# Initial kernel scaffold; baseline (speedup 1.0000x reference)
import jax
import jax.numpy as jnp
from jax.experimental import pallas as pl


def kernel(sequence, table):
    raise NotImplementedError("write your pallas kernel here")


# speedup vs baseline: 3.4502x; 3.4502x over previous
import jax
import jax.numpy as jnp
from jax.experimental import pallas as pl


def kernel(sequence, table):
    batch, seq_len = sequence.shape
    dim = table.shape[1]
    blk = 512

    def body(t_ref, o_ref):
        o_ref[...] = t_ref[...][None]

    out = pl.pallas_call(
        body,
        grid=(seq_len // blk, batch),
        in_specs=[pl.BlockSpec((blk, dim), lambda i, b: (i, 0))],
        out_specs=pl.BlockSpec((1, blk, dim), lambda i, b: (b, i, 0)),
        out_shape=jax.ShapeDtypeStruct((batch, seq_len, dim), table.dtype),
    )(table)
    return out


# speedup vs baseline: 3.6653x; 1.0623x over previous
import functools

import jax
import jax.numpy as jnp
from jax import lax
from jax.experimental import pallas as pl
from jax.experimental.pallas import tpu as pltpu
from jax.experimental.pallas import tpu_sc as plsc

NC, NS = 2, 16
NW = NC * NS


def _make_sc_kernel(batch, seq_len, dim, dtype):
    rows_per_w = seq_len // NW
    chunk = min(64, rows_per_w)
    n_chunks = rows_per_w // chunk
    mesh = plsc.VectorSubcoreMesh(core_axis_name="c", subcore_axis_name="s")

    @functools.partial(
        pl.kernel,
        mesh=mesh,
        out_type=jax.ShapeDtypeStruct((batch, seq_len, dim), dtype),
        scratch_types=[
            pltpu.VMEM((chunk, dim), dtype),
            pltpu.SemaphoreType.DMA,
        ],
    )
    def sc_kernel(table_hbm, out_hbm, buf, sem):
        wid = lax.axis_index("s") * NC + lax.axis_index("c")
        base = wid * rows_per_w
        for c in range(n_chunks):
            off = base + c * chunk
            pltpu.sync_copy(table_hbm.at[pl.ds(off, chunk)], buf)
            copies = [
                pltpu.async_copy(buf, out_hbm.at[b, pl.ds(off, chunk)], sem)
                for b in range(batch)
            ]
            for cp in copies:
                cp.wait()

    return sc_kernel


def kernel(sequence, table):
    batch, seq_len = sequence.shape
    dim = table.shape[1]
    return _make_sc_kernel(batch, seq_len, dim, table.dtype)(table)
